# baseline (speedup 1.0000x reference)
import jax
import jax.numpy as jnp
from jax import lax
from jax.experimental import pallas as pl
from jax.experimental.pallas import tpu as pltpu
from jax.experimental.pallas import tpu_sc as plsc

N = 10000
E = 320000
D = 128
NC = 2
NS = 16
NW = NC * NS
CHUNK = 128
NCHUNKS = 80
EDGES_PER_TILE = CHUNK * NCHUNKS
EP = NW * EDGES_PER_TILE
DEG_CHUNKS = 2 * NCHUNKS
ROWS_PER_SUB = 624
TAIL_BASE = ROWS_PER_SUB * NS
TAIL_ROWS = N - TAIL_BASE
NVREG = N // 16

_VECTOR_MESH = plsc.VectorSubcoreMesh(
    core_axis_name="c", subcore_axis_name="s", num_cores=NC, num_subcores=NS)

_SC_PARAMS = pltpu.CompilerParams(needs_layout_passes=False)


def _sc_body(xw, rowp, colp, ea0p, zeros1, zeros2, accp, degh,
             dinv_v, sbuf, ir0, ir1, ic0, ic1, ie0, ie1, rows0, rows1,
             g0, g1, r0, r1, c0, c1, e0, e1, acc_sh, deg_sh):
    c = lax.axis_index("c")
    s = lax.axis_index("s")
    wid = s * NC + c

    pltpu.sync_copy(zeros2.at[pl.ds(s * ROWS_PER_SUB, ROWS_PER_SUB)],
                    acc_sh.at[pl.ds(s * ROWS_PER_SUB, ROWS_PER_SUB)])

    @pl.when(s == NS - 1)
    def _():
        pltpu.sync_copy(zeros2.at[pl.ds(TAIL_BASE, TAIL_ROWS)],
                        acc_sh.at[pl.ds(TAIL_BASE, TAIL_ROWS)])

    plsc.subcore_barrier()

    dbase = s * DEG_CHUNKS * CHUNK

    def dld(u, ibuf, isem, vbuf, vsem):
        pltpu.async_copy(colp.at[pl.ds(dbase + u * CHUNK, CHUNK)], ibuf, isem)
        pltpu.async_copy(ea0p.at[pl.ds(dbase + u * CHUNK, CHUNK)], vbuf, vsem)

    def dld_wait(u, ibuf, isem, vbuf, vsem):
        pltpu.make_async_copy(
            colp.at[pl.ds(dbase + u * CHUNK, CHUNK)], ibuf, isem).wait()
        pltpu.make_async_copy(
            ea0p.at[pl.ds(dbase + u * CHUNK, CHUNK)], vbuf, vsem).wait()

    def dhalf(u, ibuf, isem, vbuf, vsem):
        dld_wait(u, ibuf, isem, vbuf, vsem)
        for k in range(CHUNK // 16):
            sl = pl.ds(k * 16, 16)
            vbuf[sl] = jnp.abs(vbuf[sl])
        pltpu.sync_copy(vbuf, deg_sh.at[ibuf], add=True)

        @pl.when(u + 2 < DEG_CHUNKS)
        def _():
            dld(u + 2, ibuf, isem, vbuf, vsem)

    @pl.when(s == 0)
    def _():
        pltpu.sync_copy(zeros1, deg_sh)

    plsc.subcore_barrier()

    dld(0, ir0, r0, ie0, e0)
    dld(1, ir1, r1, ie1, e1)

    @pl.loop(0, DEG_CHUNKS, step=2)
    def _(u):
        dhalf(u, ir0, r0, ie0, e0)
        dhalf(u + 1, ir1, r1, ie1, e1)

    plsc.subcore_barrier()

    @pl.when((s == 0) & (c == 0))
    def _():
        pltpu.sync_copy(deg_sh, degh)

    pltpu.sync_copy(deg_sh, dinv_v)

    @plsc.parallel_loop(0, NVREG, unroll=5)
    def _(i):
        sl = pl.ds(i * 16, 16)
        v = dinv_v[sl] + 1.0
        xi = plsc.bitcast(v, jnp.int32)
        xi = jnp.int32(0x5F3759DF) - lax.shift_right_arithmetic(
            xi, jnp.int32(1))
        xr = plsc.bitcast(xi, jnp.float32)
        for _ in range(3):
            xr = xr * (1.5 - 0.5 * v * xr * xr)
        dinv_v[sl] = xr


    ebase = wid * EDGES_PER_TILE

    def scale(t, buf):
        @plsc.parallel_loop(0, CHUNK, unroll=4)
        def _(j):
            sc_v = plsc.load_gather(sbuf, [jnp.full((16,), j, jnp.int32)])
            for k in range(D // 16):
                sl = pl.ds(k * 16, 16)
                buf[j, sl] = buf[j, sl] * sc_v

    def mk_sbuf(irA, ieA):
        for g in range(CHUNK // 16):
            sl = pl.ds(g * 16, 16)
            dv = plsc.load_gather(dinv_v, [irA[sl]])
            sbuf[sl] = jnp.abs(ieA[sl]) * dv

    def ld(t, buf, sem, src):
        pltpu.async_copy(src.at[pl.ds(ebase + t * CHUNK, CHUNK)], buf, sem)

    def ld_wait(t, buf, sem, src):
        pltpu.make_async_copy(
            src.at[pl.ds(ebase + t * CHUNK, CHUNK)], buf, sem).wait()

    def half(t, irA, irB, icA, ieA, rowsA, rowsB, gA, gB, rA, rB, cA, eA):
        pltpu.make_async_copy(xw.at[irA], rowsA, gA).wait()
        ld_wait(t, ieA, eA, ea0p)
        mk_sbuf(irA, ieA)

        @pl.when(t + 2 < NCHUNKS)
        def _():
            ld(t + 2, irA, rA, rowp)
            ld(t + 2, ieA, eA, ea0p)

        @pl.when(t + 1 < NCHUNKS)
        def _():
            ld_wait(t + 1, irB, rB, rowp)
            pltpu.async_copy(xw.at[irB], rowsB, gB)

        scale(t, rowsA)
        ld_wait(t, icA, cA, colp)
        pltpu.sync_copy(rowsA, acc_sh.at[icA], add=True)

        @pl.when(t + 2 < NCHUNKS)
        def _():
            ld(t + 2, icA, cA, colp)

    ld(0, ir0, r0, rowp)
    ld(1, ir1, r1, rowp)
    ld(0, ie0, e0, ea0p)
    ld(1, ie1, e1, ea0p)
    ld(0, ic0, c0, colp)
    ld(1, ic1, c1, colp)
    ld_wait(0, ir0, r0, rowp)
    pltpu.async_copy(xw.at[ir0], rows0, g0)

    @pl.loop(0, NCHUNKS, step=2)
    def _(t):
        half(t, ir0, ir1, ic0, ie0, rows0, rows1, g0, g1, r0, r1, c0, e0)
        half(t + 1, ir1, ir0, ic1, ie1, rows1, rows0, g1, g0, r1, r0, c1, e1)

    plsc.subcore_barrier()

    pltpu.sync_copy(acc_sh.at[pl.ds(s * ROWS_PER_SUB, ROWS_PER_SUB)],
                    accp.at[c, pl.ds(s * ROWS_PER_SUB, ROWS_PER_SUB)])

    @pl.when(s == NS - 1)
    def _():
        pltpu.sync_copy(acc_sh.at[pl.ds(TAIL_BASE, TAIL_ROWS)],
                        accp.at[c, pl.ds(TAIL_BASE, TAIL_ROWS)])


def _sc_pass(xw, rowp, colp, ea0p, zeros1, zeros2):
    k = pl.kernel(
        _sc_body,
        out_type=[jax.ShapeDtypeStruct((NC, N, D), jnp.float32),
                  jax.ShapeDtypeStruct((N,), jnp.float32)],
        mesh=_VECTOR_MESH,
        scratch_types=[
            pltpu.VMEM((N,), jnp.float32),
            pltpu.VMEM((CHUNK,), jnp.float32),
            pltpu.VMEM((CHUNK,), jnp.int32),
            pltpu.VMEM((CHUNK,), jnp.int32),
            pltpu.VMEM((CHUNK,), jnp.int32),
            pltpu.VMEM((CHUNK,), jnp.int32),
            pltpu.VMEM((CHUNK,), jnp.float32),
            pltpu.VMEM((CHUNK,), jnp.float32),
            pltpu.VMEM((CHUNK, D), jnp.float32),
            pltpu.VMEM((CHUNK, D), jnp.float32),
            pltpu.SemaphoreType.DMA,
            pltpu.SemaphoreType.DMA,
            pltpu.SemaphoreType.DMA,
            pltpu.SemaphoreType.DMA,
            pltpu.SemaphoreType.DMA,
            pltpu.SemaphoreType.DMA,
            pltpu.SemaphoreType.DMA,
            pltpu.SemaphoreType.DMA,
            pltpu.VMEM_SHARED((N, D), jnp.float32),
            pltpu.VMEM_SHARED((N,), jnp.float32),
        ],
        compiler_params=_SC_PARAMS,
    )
    return k(xw, rowp, colp, ea0p, zeros1, zeros2)


_BLK = 400


def _mm_body(x_ref, w_ref, o_ref):
    o_ref[...] = lax.dot_general(
        x_ref[...], w_ref[...], (((1,), (1,)), ((), ())),
        preferred_element_type=jnp.float32)


def _matmul(x, W):
    return pl.pallas_call(
        _mm_body,
        grid=(N // _BLK,),
        in_specs=[pl.BlockSpec((_BLK, D), lambda i: (i, 0)),
                  pl.BlockSpec((D, D), lambda i: (0, 0))],
        out_specs=pl.BlockSpec((_BLK, D), lambda i: (i, 0)),
        out_shape=jax.ShapeDtypeStruct((N, D), jnp.float32),
    )(x, W)


def _post_body(a0_ref, a1_ref, xw_ref, deg_ref, b_ref, lw_ref, lb_ref, o_ref):
    dinv = lax.rsqrt(deg_ref[...] + 1.0)
    pre = dinv * (a0_ref[...] + a1_ref[...] + dinv * xw_ref[...]) + b_ref[...]
    act = jnp.where(pre >= 0, pre, 0.01 * pre)
    mu = jnp.mean(act, axis=1, keepdims=True)
    zc = act - mu
    var = jnp.mean(zc * zc, axis=1, keepdims=True)
    o_ref[...] = zc * lax.rsqrt(var + 1e-5) * lw_ref[...] + lb_ref[...]


def _post(a0, a1, xw, deg, b, lw, lb):
    return pl.pallas_call(
        _post_body,
        grid=(N // _BLK,),
        in_specs=[pl.BlockSpec((_BLK, D), lambda i: (i, 0)),
                  pl.BlockSpec((_BLK, D), lambda i: (i, 0)),
                  pl.BlockSpec((_BLK, D), lambda i: (i, 0)),
                  pl.BlockSpec((_BLK, 1), lambda i: (i, 0)),
                  pl.BlockSpec((1, D), lambda i: (0, 0)),
                  pl.BlockSpec((1, D), lambda i: (0, 0)),
                  pl.BlockSpec((1, D), lambda i: (0, 0))],
        out_specs=pl.BlockSpec((_BLK, D), lambda i: (i, 0)),
        out_shape=jax.ShapeDtypeStruct((N, D), jnp.float32),
    )(a0, a1, xw, deg, b, lw, lb)


def kernel(x, edge_attr, W, b, ln_w, ln_b, edge_index):
    ei = edge_index.astype(jnp.int32)
    row = ei[0]
    col = ei[1]
    ea0 = edge_attr[:, 0]
    pad = EP - E
    padi = jnp.arange(pad, dtype=jnp.int32) % N
    rowp = jnp.concatenate([row, padi])
    colp = jnp.concatenate([col, padi])
    ea0p = jnp.concatenate([ea0, jnp.zeros((pad,), jnp.float32)])
    zeros1 = jnp.zeros((N,), jnp.float32)
    zeros2 = jnp.zeros((N, D), jnp.float32)

    xw = _matmul(x, W)
    accp, deg = _sc_pass(xw, rowp, colp, ea0p, zeros1, zeros2)
    return _post(accp[0], accp[1], xw, deg.reshape(N, 1), b.reshape(1, D),
                 ln_w.reshape(1, D), ln_b.reshape(1, D))


# speedup vs baseline: 1.1440x; 1.1440x over previous
import jax
import jax.numpy as jnp
from jax import lax
from jax.experimental import pallas as pl
from jax.experimental.pallas import tpu as pltpu
from jax.experimental.pallas import tpu_sc as plsc

N = 10000
E = 320000
D = 128
NC = 2
NS = 16
NW = NC * NS
CHUNK = 128
NCHUNKS = 80
EDGES_PER_TILE = CHUNK * NCHUNKS
EP = NW * EDGES_PER_TILE
ROWS_PER_SUB = 624
TAIL_BASE = ROWS_PER_SUB * NS
TAIL_ROWS = N - TAIL_BASE

_VECTOR_MESH = plsc.VectorSubcoreMesh(
    core_axis_name="c", subcore_axis_name="s", num_cores=NC, num_subcores=NS)

_SC_PARAMS = pltpu.CompilerParams(needs_layout_passes=False)


def _deg_body(col2, ea0p, zeros1, degp, idx_v, val_v, acc_sh):
    c = lax.axis_index("c")
    s = lax.axis_index("s")
    wid = s * NC + c
    ebase = wid * EDGES_PER_TILE

    @pl.when(s == 0)
    def _():
        pltpu.sync_copy(zeros1, acc_sh)

    pltpu.sync_copy(col2.at[pl.ds(wid * NCHUNKS, NCHUNKS)], idx_v)
    pltpu.sync_copy(ea0p.at[pl.ds(ebase, EDGES_PER_TILE)], val_v)

    @pl.loop(0, EDGES_PER_TILE, step=16)
    def _(e):
        sl = pl.ds(e, 16)
        val_v[sl] = jnp.abs(val_v[sl])

    plsc.subcore_barrier()

    @pl.loop(0, NCHUNKS)
    def _(t):
        pltpu.sync_copy(val_v.at[pl.ds(t * CHUNK, CHUNK)],
                        acc_sh.at[idx_v.at[t]],
                        add=True)

    plsc.subcore_barrier()

    @pl.when(s == 0)
    def _():
        pltpu.sync_copy(acc_sh, degp.at[c])


def _deg_partials(col2, ea0p, zeros1):
    k = pl.kernel(
        _deg_body,
        out_type=jax.ShapeDtypeStruct((NC, N), jnp.float32),
        mesh=_VECTOR_MESH,
        scratch_types=[
            pltpu.VMEM((NCHUNKS, CHUNK), jnp.int32),
            pltpu.VMEM((EDGES_PER_TILE,), jnp.float32),
            pltpu.VMEM_SHARED((N,), jnp.float32),
        ],
        compiler_params=_SC_PARAMS,
    )
    return k(col2, ea0p, zeros1)


def _msg_body(y, rowp, colp, ea0p, zeros2, accp,
              ew_v, ir0, ir1, ic0, ic1, rows0, rows1,
              g0, g1, r0, r1, c0, c1, acc_sh):
    c = lax.axis_index("c")
    s = lax.axis_index("s")
    wid = s * NC + c
    ebase = wid * EDGES_PER_TILE

    pltpu.sync_copy(zeros2.at[pl.ds(s * ROWS_PER_SUB, ROWS_PER_SUB)],
                    acc_sh.at[pl.ds(s * ROWS_PER_SUB, ROWS_PER_SUB)])

    @pl.when(s == NS - 1)
    def _():
        pltpu.sync_copy(zeros2.at[pl.ds(TAIL_BASE, TAIL_ROWS)],
                        acc_sh.at[pl.ds(TAIL_BASE, TAIL_ROWS)])

    pltpu.sync_copy(ea0p.at[pl.ds(ebase, EDGES_PER_TILE)], ew_v)

    @pl.loop(0, EDGES_PER_TILE, step=16)
    def _(e):
        sl = pl.ds(e, 16)
        ew_v[sl] = jnp.abs(ew_v[sl])

    plsc.subcore_barrier()

    def scale(t, buf):
        @plsc.parallel_loop(0, CHUNK, unroll=8)
        def _(j):
            sc_v = plsc.load_gather(
                ew_v, [jnp.full((16,), t * CHUNK + j, jnp.int32)])
            for k in range(D // 16):
                sl = pl.ds(k * 16, 16)
                buf[j, sl] = buf[j, sl] * sc_v

    def ir_load(t, buf, sem):
        pltpu.async_copy(rowp.at[pl.ds(ebase + t * CHUNK, CHUNK)], buf, sem)

    def ir_wait(t, buf, sem):
        pltpu.make_async_copy(
            rowp.at[pl.ds(ebase + t * CHUNK, CHUNK)], buf, sem).wait()

    def ic_load(t, buf, sem):
        pltpu.async_copy(colp.at[pl.ds(ebase + t * CHUNK, CHUNK)], buf, sem)

    def ic_wait(t, buf, sem):
        pltpu.make_async_copy(
            colp.at[pl.ds(ebase + t * CHUNK, CHUNK)], buf, sem).wait()

    def half(t, irA, irB, icA, rowsA, rowsB, gA, gB, rA, rB, cA):
        pltpu.make_async_copy(y.at[irA], rowsA, gA).wait()

        @pl.when(t + 2 < NCHUNKS)
        def _():
            ir_load(t + 2, irA, rA)

        @pl.when(t + 1 < NCHUNKS)
        def _():
            ir_wait(t + 1, irB, rB)
            pltpu.async_copy(y.at[irB], rowsB, gB)

        scale(t, rowsA)
        ic_wait(t, icA, cA)
        pltpu.sync_copy(rowsA, acc_sh.at[icA], add=True)

        @pl.when(t + 2 < NCHUNKS)
        def _():
            ic_load(t + 2, icA, cA)

    ir_load(0, ir0, r0)
    ir_load(1, ir1, r1)
    ic_load(0, ic0, c0)
    ic_load(1, ic1, c1)
    ir_wait(0, ir0, r0)
    pltpu.async_copy(y.at[ir0], rows0, g0)

    @pl.loop(0, NCHUNKS, step=2)
    def _(t):
        half(t, ir0, ir1, ic0, rows0, rows1, g0, g1, r0, r1, c0)
        half(t + 1, ir1, ir0, ic1, rows1, rows0, g1, g0, r1, r0, c1)

    plsc.subcore_barrier()

    pltpu.sync_copy(acc_sh.at[pl.ds(s * ROWS_PER_SUB, ROWS_PER_SUB)],
                    accp.at[c, pl.ds(s * ROWS_PER_SUB, ROWS_PER_SUB)])

    @pl.when(s == NS - 1)
    def _():
        pltpu.sync_copy(acc_sh.at[pl.ds(TAIL_BASE, TAIL_ROWS)],
                        accp.at[c, pl.ds(TAIL_BASE, TAIL_ROWS)])


def _msg_partials(y, rowp, colp, ea0p, zeros2):
    k = pl.kernel(
        _msg_body,
        out_type=jax.ShapeDtypeStruct((NC, N, D), jnp.float32),
        mesh=_VECTOR_MESH,
        scratch_types=[
            pltpu.VMEM((EDGES_PER_TILE,), jnp.float32),
            pltpu.VMEM((CHUNK,), jnp.int32),
            pltpu.VMEM((CHUNK,), jnp.int32),
            pltpu.VMEM((CHUNK,), jnp.int32),
            pltpu.VMEM((CHUNK,), jnp.int32),
            pltpu.VMEM((CHUNK, D), jnp.float32),
            pltpu.VMEM((CHUNK, D), jnp.float32),
            pltpu.SemaphoreType.DMA,
            pltpu.SemaphoreType.DMA,
            pltpu.SemaphoreType.DMA,
            pltpu.SemaphoreType.DMA,
            pltpu.SemaphoreType.DMA,
            pltpu.SemaphoreType.DMA,
            pltpu.VMEM_SHARED((N, D), jnp.float32),
        ],
        compiler_params=_SC_PARAMS,
    )
    return k(y, rowp, colp, ea0p, zeros2)


_BLK = 400


def _y_body(x_ref, w_ref, d0_ref, d1_ref, y_ref, dinv_ref):
    xw = lax.dot_general(
        x_ref[...], w_ref[...], (((1,), (1,)), ((), ())),
        preferred_element_type=jnp.float32)
    deg = d0_ref[...] + d1_ref[...] + 1.0
    dinv = lax.rsqrt(deg)
    dinv_ref[...] = dinv
    y_ref[...] = xw * dinv


def _y_dinv(x, W, d0, d1):
    return pl.pallas_call(
        _y_body,
        grid=(N // _BLK,),
        in_specs=[pl.BlockSpec((_BLK, D), lambda i: (i, 0)),
                  pl.BlockSpec((D, D), lambda i: (0, 0)),
                  pl.BlockSpec((_BLK, 1), lambda i: (i, 0)),
                  pl.BlockSpec((_BLK, 1), lambda i: (i, 0))],
        out_specs=[pl.BlockSpec((_BLK, D), lambda i: (i, 0)),
                   pl.BlockSpec((_BLK, 1), lambda i: (i, 0))],
        out_shape=[jax.ShapeDtypeStruct((N, D), jnp.float32),
                   jax.ShapeDtypeStruct((N, 1), jnp.float32)],
    )(x, W, d0, d1)


def _post_body(acc_ref, y_ref, dinv_ref, b_ref, lw_ref, lb_ref, o_ref):
    pre = dinv_ref[...] * (acc_ref[0] + acc_ref[1] + y_ref[...]) + b_ref[...]
    act = jnp.where(pre >= 0, pre, 0.01 * pre)
    mu = jnp.mean(act, axis=1, keepdims=True)
    zc = act - mu
    var = jnp.mean(zc * zc, axis=1, keepdims=True)
    o_ref[...] = zc * lax.rsqrt(var + 1e-5) * lw_ref[...] + lb_ref[...]


def _post(accp, y, dinv, b, lw, lb):
    return pl.pallas_call(
        _post_body,
        grid=(N // _BLK,),
        in_specs=[pl.BlockSpec((NC, _BLK, D), lambda i: (0, i, 0)),
                  pl.BlockSpec((_BLK, D), lambda i: (i, 0)),
                  pl.BlockSpec((_BLK, 1), lambda i: (i, 0)),
                  pl.BlockSpec((1, D), lambda i: (0, 0)),
                  pl.BlockSpec((1, D), lambda i: (0, 0)),
                  pl.BlockSpec((1, D), lambda i: (0, 0))],
        out_specs=pl.BlockSpec((_BLK, D), lambda i: (i, 0)),
        out_shape=jax.ShapeDtypeStruct((N, D), jnp.float32),
    )(accp, y, dinv, b, lw, lb)


def kernel(x, edge_attr, W, b, ln_w, ln_b, edge_index):
    ei = edge_index.astype(jnp.int32)
    row = ei[0]
    col = ei[1]
    ea0 = edge_attr[:, 0]
    pad = EP - E
    padi = jnp.arange(pad, dtype=jnp.int32) % N
    rowp = jnp.concatenate([row, padi])
    colp = jnp.concatenate([col, padi])
    ea0p = jnp.concatenate([ea0, jnp.zeros((pad,), jnp.float32)])
    zeros1 = jnp.zeros((N,), jnp.float32)
    zeros2 = jnp.zeros((N, D), jnp.float32)

    col2 = colp.reshape(NW * NCHUNKS, CHUNK)
    degp = _deg_partials(col2, ea0p, zeros1)
    y, dinv = _y_dinv(x, W, degp[0].reshape(N, 1), degp[1].reshape(N, 1))
    accp = _msg_partials(y, rowp, colp, ea0p, zeros2)
    return _post(accp, y, dinv, b.reshape(1, D),
                 ln_w.reshape(1, D), ln_b.reshape(1, D))
